# baseline (speedup 1.0000x reference)
import functools

import jax
import jax.numpy as jnp
from jax import lax
from jax.experimental import pallas as pl
from jax.experimental.pallas import tpu as pltpu
from jax.experimental.pallas import tpu_sc as plsc

N_CAT = 26
VOCAB = 100000
EMB = 5
N_NUM = 13
B = 4096

NC = 2
NS = 16
NW = NC * NS

NELEM = B * N_CAT * EMB
E_PER_W = NELEM // NW
CHUNK = 128
N_CHUNK = E_PER_W // CHUNK
GROUP = 13


@functools.cache
def _make_sc_gather():
    @functools.partial(
        pl.kernel,
        out_type=jax.ShapeDtypeStruct((NELEM,), jnp.float32),
        mesh=plsc.VectorSubcoreMesh(core_axis_name="c", subcore_axis_name="s"),
        scratch_types=[
            pltpu.VMEM((E_PER_W,), jnp.int32),
            pltpu.VMEM((E_PER_W,), jnp.float32),
            pltpu.SemaphoreType.DMA,
        ],
    )
    def _sc_gather(table_hbm, idx_hbm, out_hbm, idx_v, vals_v, sem):
        wid = lax.axis_index("s") * NC + lax.axis_index("c")
        base = wid * E_PER_W
        pltpu.sync_copy(idx_hbm.at[pl.ds(base, E_PER_W)], idx_v)

        @pl.loop(0, N_CHUNK, step=GROUP)
        def _(g):
            for j in range(GROUP):
                o = (g + j) * CHUNK
                pltpu.async_copy(
                    table_hbm.at[idx_v.at[pl.ds(o, CHUNK)]],
                    vals_v.at[pl.ds(o, CHUNK)],
                    sem,
                )
            for j in range(GROUP):
                o = (g + j) * CHUNK
                pltpu.make_async_copy(
                    table_hbm.at[idx_v.at[pl.ds(o, CHUNK)]],
                    vals_v.at[pl.ds(o, CHUNK)],
                    sem,
                ).wait()

        pltpu.sync_copy(vals_v, out_hbm.at[pl.ds(base, E_PER_W)])

    return _sc_gather


def _tc_body(cat_ref, num_ref, w_ref, b_ref, out_ref):
    num_out = lax.dot_general(
        num_ref[...], w_ref[...],
        (((1,), (1,)), ((), ())),
        preferred_element_type=jnp.float32,
    ) + b_ref[...]
    out_ref[...] = jnp.concatenate([cat_ref[...], num_out], axis=1)


def kernel(cate_indices, num_values, emb_tables, W, b):
    row = (
        cate_indices.astype(jnp.int32)
        + (jnp.arange(N_CAT, dtype=jnp.int32) * VOCAB)[:, None]
    ).T.reshape(-1, 1)
    flat_idx = (
        row + jnp.arange(EMB, dtype=jnp.int32) * (N_CAT * VOCAB)
    ).reshape(-1)
    table_flat = jnp.transpose(emb_tables, (2, 0, 1)).reshape(-1)

    cat = _make_sc_gather()(table_flat, flat_idx).reshape(B, N_CAT * EMB)

    blk = 1024
    out = pl.pallas_call(
        _tc_body,
        grid=(B // blk,),
        in_specs=[
            pl.BlockSpec((blk, N_CAT * EMB), lambda i: (i, 0)),
            pl.BlockSpec((blk, N_NUM), lambda i: (i, 0)),
            pl.BlockSpec((N_NUM, N_NUM), lambda i: (0, 0)),
            pl.BlockSpec((1, N_NUM), lambda i: (0, 0)),
        ],
        out_specs=pl.BlockSpec((blk, N_CAT * EMB + N_NUM), lambda i: (i, 0)),
        out_shape=jax.ShapeDtypeStruct((B, N_CAT * EMB + N_NUM), jnp.float32),
    )(cat, num_values, W, b.reshape(1, N_NUM))
    return out


# speedup vs baseline: 4.0016x; 4.0016x over previous
import dataclasses
import functools

import jax
import jax.numpy as jnp
from jax import lax
from jax.experimental import pallas as pl
from jax.experimental.pallas import tpu as pltpu
from jax.experimental.pallas import tpu_sc as plsc

N_CAT = 26
VOCAB = 100000
EMB = 5
N_NUM = 13
B = 4096

NC = 2
NS = 16
NW = NC * NS

NROW = B * N_CAT
R_PER_W = NROW // NW
NELEM = NROW * EMB
E_PER_W = R_PER_W * EMB
CHUNK = 128
N_CHUNK = R_PER_W // CHUNK
GROUP = 13


@functools.cache
def _make_sc_gather():
    cp = pltpu.CompilerParams()
    if "needs_layout_passes" in pltpu.CompilerParams.__dataclass_fields__:
        cp = dataclasses.replace(cp, needs_layout_passes=False)

    @functools.partial(
        pl.kernel,
        compiler_params=cp,
        out_type=jax.ShapeDtypeStruct((NELEM,), jnp.float32),
        mesh=plsc.VectorSubcoreMesh(core_axis_name="c", subcore_axis_name="s"),
        scratch_types=[
            pltpu.VMEM((R_PER_W,), jnp.int32),
            pltpu.VMEM((E_PER_W,), jnp.float32),
            pltpu.VMEM((E_PER_W,), jnp.float32),
            pltpu.SemaphoreType.DMA,
        ],
    )
    def _sc_gather(t0, t1, t2, t3, t4, idx_hbm, out_hbm,
                   idx_v, vals_v, out_v, sem):
        planes = (t0, t1, t2, t3, t4)
        wid = lax.axis_index("s") * NC + lax.axis_index("c")
        base = wid * R_PER_W
        pltpu.sync_copy(idx_hbm.at[pl.ds(base, R_PER_W)], idx_v)

        for e in range(EMB):
            plane = planes[e]

            @pl.loop(0, N_CHUNK, step=GROUP)
            def _(c0, plane=plane, e=e):
                for j in range(GROUP):
                    o = (c0 + j) * CHUNK
                    pltpu.async_copy(
                        plane.at[idx_v.at[pl.ds(o, CHUNK)]],
                        vals_v.at[pl.ds(e * R_PER_W + o, CHUNK)],
                        sem,
                    )
                for j in range(GROUP):
                    o = (c0 + j) * CHUNK
                    pltpu.make_async_copy(
                        plane.at[idx_v.at[pl.ds(o, CHUNK)]],
                        vals_v.at[pl.ds(e * R_PER_W + o, CHUNK)],
                        sem,
                    ).wait()

        iota = lax.iota(jnp.int32, 16)
        pat = [iota * EMB + e for e in range(EMB)]

        @pl.loop(0, R_PER_W // 16)
        def _(t):
            for e in range(EMB):
                v = vals_v[pl.ds(e * R_PER_W + t * 16, 16)]
                plsc.store_scatter(out_v, [pat[e] + t * 80], v)

        pltpu.sync_copy(out_v, out_hbm.at[pl.ds(base * EMB, E_PER_W)])

    return _sc_gather


def _tc_body(cat_ref, num_ref, w_ref, b_ref, out_ref):
    num_out = lax.dot_general(
        num_ref[...], w_ref[...],
        (((1,), (1,)), ((), ())),
        preferred_element_type=jnp.float32,
    ) + b_ref[...]
    out_ref[...] = jnp.concatenate([cat_ref[...], num_out], axis=1)


def kernel(cate_indices, num_values, emb_tables, W, b):
    row = (
        cate_indices.astype(jnp.int32)
        + (jnp.arange(N_CAT, dtype=jnp.int32) * VOCAB)[:, None]
    ).T.reshape(-1)
    t = jnp.transpose(emb_tables, (2, 0, 1))
    planes = [t[e].reshape(-1) for e in range(EMB)]

    cat = _make_sc_gather()(*planes, row).reshape(B, N_CAT * EMB)

    blk = 1024
    out = pl.pallas_call(
        _tc_body,
        grid=(B // blk,),
        in_specs=[
            pl.BlockSpec((blk, N_CAT * EMB), lambda i: (i, 0)),
            pl.BlockSpec((blk, N_NUM), lambda i: (i, 0)),
            pl.BlockSpec((N_NUM, N_NUM), lambda i: (0, 0)),
            pl.BlockSpec((1, N_NUM), lambda i: (0, 0)),
        ],
        out_specs=pl.BlockSpec((blk, N_CAT * EMB + N_NUM), lambda i: (i, 0)),
        out_shape=jax.ShapeDtypeStruct((B, N_CAT * EMB + N_NUM), jnp.float32),
    )(cat, num_values, W, b.reshape(1, N_NUM))
    return out


# speedup vs baseline: 4.0251x; 1.0059x over previous
import dataclasses
import functools

import jax
import jax.numpy as jnp
from jax import lax
from jax.experimental import pallas as pl
from jax.experimental.pallas import tpu as pltpu
from jax.experimental.pallas import tpu_sc as plsc

N_CAT = 26
VOCAB = 100000
EMB = 5
N_NUM = 13
B = 4096

NC = 2
NS = 16
NW = NC * NS

NROW = B * N_CAT
R_PER_W = NROW // NW
NELEM = NROW * EMB
E_PER_W = R_PER_W * EMB
CHUNK = 128
N_CHUNK = R_PER_W // CHUNK
GROUP = 13


@functools.cache
def _make_sc_gather():
    cp = pltpu.CompilerParams()
    if "needs_layout_passes" in pltpu.CompilerParams.__dataclass_fields__:
        cp = dataclasses.replace(cp, needs_layout_passes=False)

    @functools.partial(
        pl.kernel,
        compiler_params=cp,
        out_type=jax.ShapeDtypeStruct((NELEM,), jnp.float32),
        mesh=plsc.VectorSubcoreMesh(core_axis_name="c", subcore_axis_name="s"),
        scratch_types=[
            pltpu.VMEM((N_CAT, B // NW), jnp.int32),
            pltpu.VMEM((R_PER_W,), jnp.int32),
            pltpu.VMEM((E_PER_W,), jnp.float32),
            pltpu.VMEM((E_PER_W,), jnp.float32),
            pltpu.SemaphoreType.DMA,
        ],
    )
    def _sc_gather(t0, t1, t2, t3, t4, cate_hbm, out_hbm,
                   cate_v, idx_v, vals_v, out_v, sem):
        planes = (t0, t1, t2, t3, t4)
        wid = lax.axis_index("s") * NC + lax.axis_index("c")
        base = wid * R_PER_W
        b0 = wid * (B // NW)
        pltpu.sync_copy(cate_hbm.at[:, pl.ds(b0, B // NW)], cate_v)

        iota16 = lax.iota(jnp.int32, 16)
        pats = []
        for p in range(13):
            s = (16 * p) % N_CAT
            db = jnp.where(iota16 >= (N_CAT - s), 1, 0).astype(jnp.int32)
            i_p = s + iota16 - N_CAT * db
            bb = (16 * p) // N_CAT
            pats.append((i_p, i_p * VOCAB, db + bb))

        @pl.loop(0, R_PER_W // 208)
        def _(q):
            for p in range(13):
                i_p, ivo_p, db_p = pats[p]
                v = plsc.load_gather(cate_v, [i_p, db_p + q * 8])
                idx_v[pl.ds(q * 208 + p * 16, 16)] = v + ivo_p

        for e in range(EMB):
            plane = planes[e]

            @pl.loop(0, N_CHUNK, step=GROUP)
            def _(c0, plane=plane, e=e):
                for j in range(GROUP):
                    o = (c0 + j) * CHUNK
                    pltpu.async_copy(
                        plane.at[idx_v.at[pl.ds(o, CHUNK)]],
                        vals_v.at[pl.ds(e * R_PER_W + o, CHUNK)],
                        sem,
                    )
                for j in range(GROUP):
                    o = (c0 + j) * CHUNK
                    pltpu.make_async_copy(
                        plane.at[idx_v.at[pl.ds(o, CHUNK)]],
                        vals_v.at[pl.ds(e * R_PER_W + o, CHUNK)],
                        sem,
                    ).wait()

        iota = lax.iota(jnp.int32, 16)
        pat = [iota * EMB + e for e in range(EMB)]

        @pl.loop(0, R_PER_W // 16)
        def _(t):
            for e in range(EMB):
                v = vals_v[pl.ds(e * R_PER_W + t * 16, 16)]
                plsc.store_scatter(out_v, [pat[e] + t * 80], v)

        pltpu.sync_copy(out_v, out_hbm.at[pl.ds(base * EMB, E_PER_W)])

    return _sc_gather


def _tc_body(cat_ref, num_ref, w_ref, b_ref, out_ref):
    num_out = lax.dot_general(
        num_ref[...], w_ref[...],
        (((1,), (1,)), ((), ())),
        preferred_element_type=jnp.float32,
    ) + b_ref[...]
    out_ref[...] = jnp.concatenate([cat_ref[...], num_out], axis=1)


def kernel(cate_indices, num_values, emb_tables, W, b):
    t = jnp.transpose(emb_tables, (2, 0, 1))
    planes = [t[e].reshape(-1) for e in range(EMB)]

    cat = _make_sc_gather()(
        *planes, cate_indices.astype(jnp.int32)
    ).reshape(B, N_CAT * EMB)

    blk = 1024
    out = pl.pallas_call(
        _tc_body,
        grid=(B // blk,),
        in_specs=[
            pl.BlockSpec((blk, N_CAT * EMB), lambda i: (i, 0)),
            pl.BlockSpec((blk, N_NUM), lambda i: (i, 0)),
            pl.BlockSpec((N_NUM, N_NUM), lambda i: (0, 0)),
            pl.BlockSpec((1, N_NUM), lambda i: (0, 0)),
        ],
        out_specs=pl.BlockSpec((blk, N_CAT * EMB + N_NUM), lambda i: (i, 0)),
        out_shape=jax.ShapeDtypeStruct((B, N_CAT * EMB + N_NUM), jnp.float32),
    )(cat, num_values, W, b.reshape(1, N_NUM))
    return out
